# baseline (speedup 1.0000x reference)
import functools

import jax
import jax.numpy as jnp
from jax import lax
from jax.experimental import pallas as pl
from jax.experimental.pallas import tpu as pltpu
from jax.experimental.pallas import tpu_sc as plsc

N_EXPERTS = 8
D_MODEL = 1024
EXPERT_FF = 512
TOP_K = 2
T_STEPS = 4
N_TOK = 2048
BETA = 0.9
V_TH = 1.0

N_PAIRS = N_TOK * TOP_K
PB = 128
NBMAX = N_PAIRS // PB + N_EXPERTS - 1
P_PAD = NBMAX * PB

NW = 32
S_ROWS = T_STEPS * N_TOK
S_PER_W = S_ROWS // NW
S_CHUNK = 32
C_ROWS = T_STEPS * N_TOK
C_PER_W = C_ROWS // NW
C_CHUNK = 16


@functools.lru_cache(maxsize=None)
def _sc_kernels():
    mesh = plsc.VectorSubcoreMesh(
        core_axis_name="c", subcore_axis_name="s", num_cores=2,
        num_subcores=16)

    @functools.partial(
        pl.kernel,
        out_type=jax.ShapeDtypeStruct((T_STEPS * P_PAD, D_MODEL),
                                      jnp.float32),
        mesh=mesh,
        scratch_types=[
            [pltpu.VMEM((S_CHUNK,), jnp.int32)] * 2,
            [pltpu.VMEM((S_CHUNK,), jnp.int32)] * 2,
            [pltpu.VMEM((S_CHUNK, D_MODEL), jnp.float32)] * 2,
            [pltpu.SemaphoreType.DMA] * 2,
            [pltpu.SemaphoreType.DMA] * 2,
            [pltpu.SemaphoreType.DMA] * 2,
        ],
    )
    def sc_dispatch(x_hbm, sidx0_hbm, sidx1_hbm, out_hbm, i0_v, i1_v,
                    rows_v, lsem, s0sem, s1sem):
        wid = lax.axis_index("s") * 2 + lax.axis_index("c")
        base = wid * S_PER_W
        n_chunks = S_PER_W // S_CHUNK
        ld = [None, None]
        sc0 = [None, None]
        sc1 = [None, None]
        for c in range(n_chunks):
            b = c % 2
            o = base + c * S_CHUNK
            if sc0[b] is not None:
                sc0[b].wait()
                sc1[b].wait()
            pltpu.sync_copy(sidx0_hbm.at[pl.ds(o, S_CHUNK)], i0_v[b])
            pltpu.sync_copy(sidx1_hbm.at[pl.ds(o, S_CHUNK)], i1_v[b])
            ld[b] = pltpu.async_copy(
                x_hbm.at[pl.ds(o, S_CHUNK)], rows_v[b], lsem[b])
            if c > 0:
                p = 1 - b
                ld[p].wait()
                sc0[p] = pltpu.async_copy(
                    rows_v[p], out_hbm.at[i0_v[p]], s0sem[p])
                sc1[p] = pltpu.async_copy(
                    rows_v[p], out_hbm.at[i1_v[p]], s1sem[p])
        bl = (n_chunks - 1) % 2
        ld[bl].wait()
        pltpu.async_copy(rows_v[bl], out_hbm.at[i0_v[bl]], s0sem[bl]).wait()
        pltpu.async_copy(rows_v[bl], out_hbm.at[i1_v[bl]], s1sem[bl]).wait()
        if sc0[1 - bl] is not None:
            sc0[1 - bl].wait()
            sc1[1 - bl].wait()

    @functools.partial(
        pl.kernel,
        out_type=jax.ShapeDtypeStruct((2 * C_ROWS, D_MODEL // 4), jnp.int32),
        mesh=mesh,
        scratch_types=[
            [pltpu.VMEM((2 * C_CHUNK,), jnp.int32)] * 2,
            [pltpu.VMEM((2 * C_CHUNK, D_MODEL // 4), jnp.int32)] * 2,
            [pltpu.SemaphoreType.DMA] * 2,
            [pltpu.SemaphoreType.DMA] * 2,
        ],
    )
    def sc_combine(y_hbm, cidx_hbm, out_hbm, idx_v, rows_v, gsem, wsem):
        wid = lax.axis_index("s") * 2 + lax.axis_index("c")
        chunk = 2 * C_CHUNK
        base = wid * 2 * C_PER_W
        n_chunks = C_PER_W // C_CHUNK
        gd = [None, None]
        wd = [None, None]
        for c in range(n_chunks):
            b = c % 2
            o = base + c * chunk
            if wd[b] is not None:
                wd[b].wait()
            pltpu.sync_copy(cidx_hbm.at[pl.ds(o, chunk)], idx_v[b])
            gd[b] = pltpu.async_copy(y_hbm.at[idx_v[b]], rows_v[b], gsem[b])
            if c > 0:
                gd[1 - b].wait()
                wd[1 - b] = pltpu.async_copy(
                    rows_v[1 - b],
                    out_hbm.at[pl.ds(base + (c - 1) * chunk, chunk)],
                    wsem[1 - b])
        bl = (n_chunks - 1) % 2
        gd[bl].wait()
        pltpu.sync_copy(
            rows_v[bl],
            out_hbm.at[pl.ds(base + (n_chunks - 1) * chunk, chunk)])
        if wd[1 - bl] is not None:
            wd[1 - bl].wait()

    return sc_dispatch, sc_combine


def _add_body(ya_ref, yb_ref, w_ref, out_ref):
    w0 = w_ref[:, 0][:, None]
    w1 = w_ref[:, 1][:, None]
    ya = ya_ref[0].astype(jnp.float32)
    yb = yb_ref[0].astype(jnp.float32)
    out_ref[...] = ya * w0 + yb * w1


def _ffn_body(meta_ref, xg_ref, wup_ref, wdn_ref, y_ref, s_scratch):
    j = pl.program_id(0)
    ln = meta_ref[1, j]

    @pl.when(ln > 0)
    def _():
        xf = xg_ref[...].reshape(T_STEPS * PB, D_MODEL)
        h = lax.dot_general(
            xf, wup_ref[0], (((1,), (1,)), ((), ())),
            preferred_element_type=jnp.float32)

        v = jnp.zeros((PB, EXPERT_FF), jnp.float32)
        for t in range(T_STEPS):
            v = BETA * v + h[t * PB:(t + 1) * PB, :]
            s = (v >= V_TH).astype(jnp.float32)
            v = v - s * V_TH
            s_scratch[t * PB:(t + 1) * PB, :] = s

        o = lax.dot_general(
            s_scratch[...], wdn_ref[0], (((1,), (1,)), ((), ())),
            preferred_element_type=jnp.float32)

        v2 = jnp.zeros((PB, D_MODEL), jnp.float32)
        for t in range(T_STEPS):
            v2 = BETA * v2 + o[t * PB:(t + 1) * PB, :]
            sb = v2 >= V_TH
            v2 = v2 - sb.astype(jnp.float32) * V_TH
            y_ref[t] = sb.astype(jnp.int8)


def _routing_metadata(expert_indices):
    e_flat = expert_indices.reshape(-1).astype(jnp.int32)
    oh = (e_flat[:, None] ==
          jnp.arange(N_EXPERTS, dtype=jnp.int32)[None, :]).astype(jnp.int32)
    csum = jnp.cumsum(oh, axis=0)
    counts = csum[-1]
    local_rank = jnp.sum(oh * (csum - oh), axis=1)

    blocks_per_e = (counts + PB - 1) // PB
    blk_start = jnp.concatenate(
        [jnp.zeros((1,), jnp.int32),
         jnp.cumsum(blocks_per_e).astype(jnp.int32)])
    blk_start_p = jnp.sum(oh * blk_start[None, :N_EXPERTS], axis=1)
    slotp = blk_start_p * PB + local_rank

    n_blocks = blk_start[N_EXPERTS]
    j = jnp.arange(NBMAX, dtype=jnp.int32)
    j_eff = jnp.minimum(j, n_blocks - 1)
    blk_e = jnp.clip(
        jnp.searchsorted(blk_start, j_eff, side='right').astype(jnp.int32)
        - 1, 0, N_EXPERTS - 1)
    blk_len = jnp.clip(counts[blk_e] - (j_eff - blk_start[blk_e]) * PB,
                       0, PB)
    blk_len = jnp.where(j < n_blocks, blk_len, 0)
    meta = jnp.stack([blk_e, blk_len])

    tt = jnp.arange(T_STEPS, dtype=jnp.int32)[:, None] * P_PAD
    sidx0 = (tt + slotp[0::TOP_K][None, :]).reshape(-1)
    sidx1 = (tt + slotp[1::TOP_K][None, :]).reshape(-1)
    cidx = jnp.concatenate([sidx0, sidx1])
    return sidx0, sidx1, cidx, meta


def kernel(x, expert_indices, expert_weights, W_up, W_down):
    T, N, D = x.shape
    sidx0, sidx1, cidx, meta = _routing_metadata(expert_indices)

    sc_dispatch, sc_combine = _sc_kernels()
    xg = sc_dispatch(x.reshape(T * N, D), sidx0, sidx1)
    xg = xg.reshape(T_STEPS, P_PAD, D)

    grid_spec = pltpu.PrefetchScalarGridSpec(
        num_scalar_prefetch=1,
        grid=(NBMAX,),
        in_specs=[
            pl.BlockSpec((T_STEPS, PB, D_MODEL), lambda j, m: (0, j, 0)),
            pl.BlockSpec((1, EXPERT_FF, D_MODEL), lambda j, m: (m[0, j], 0, 0)),
            pl.BlockSpec((1, D_MODEL, EXPERT_FF), lambda j, m: (m[0, j], 0, 0)),
        ],
        out_specs=pl.BlockSpec((T_STEPS, PB, D_MODEL), lambda j, m: (0, j, 0)),
        scratch_shapes=[pltpu.VMEM((T_STEPS * PB, EXPERT_FF), jnp.float32)],
    )
    y = pl.pallas_call(
        _ffn_body,
        grid_spec=grid_spec,
        out_shape=jax.ShapeDtypeStruct((T_STEPS, P_PAD, D_MODEL), jnp.int8),
    )(meta, xg, W_up, W_down)

    y32 = lax.bitcast_convert_type(
        y.reshape(T_STEPS * P_PAD, D // 4, 4), jnp.int32)
    y2 = sc_combine(y32, cidx)
    y2 = lax.bitcast_convert_type(y2, jnp.int8)
    y2 = y2.reshape(2, C_ROWS, D)
    w_c = jnp.broadcast_to(expert_weights[None], (T_STEPS, N, TOP_K))
    w_c = w_c.reshape(C_ROWS, TOP_K)

    add_rows = 512
    out = pl.pallas_call(
        _add_body,
        grid=(C_ROWS // add_rows,),
        in_specs=[
            pl.BlockSpec((1, add_rows, D_MODEL), lambda i: (0, i, 0)),
            pl.BlockSpec((1, add_rows, D_MODEL), lambda i: (1, i, 0)),
            pl.BlockSpec((add_rows, TOP_K), lambda i: (i, 0)),
        ],
        out_specs=pl.BlockSpec((add_rows, D_MODEL), lambda i: (i, 0)),
        out_shape=jax.ShapeDtypeStruct((C_ROWS, D_MODEL), jnp.float32),
    )(y2, y2, w_c)
    return out.reshape(T, N, D)


# speedup vs baseline: 2.6404x; 2.6404x over previous
import functools

import jax
import jax.numpy as jnp
from jax import lax
from jax.experimental import pallas as pl
from jax.experimental.pallas import tpu as pltpu
from jax.experimental.pallas import tpu_sc as plsc

N_EXPERTS = 8
D_MODEL = 1024
EXPERT_FF = 512
TOP_K = 2
T_STEPS = 4
N_TOK = 2048
BETA = 0.9
V_TH = 1.0

N_PAIRS = N_TOK * TOP_K
PB = 64
NBMAX = N_PAIRS // PB + N_EXPERTS - 1
P_PAD = NBMAX * PB

NW = 32
S_ROWS = T_STEPS * N_TOK
S_PER_W = S_ROWS // NW
S_CHUNK = 32
C_ROWS = T_STEPS * N_TOK
C_PER_W = C_ROWS // NW
C_CHUNK = 16


@functools.lru_cache(maxsize=None)
def _sc_kernels():
    mesh = plsc.VectorSubcoreMesh(
        core_axis_name="c", subcore_axis_name="s", num_cores=2,
        num_subcores=16)

    @functools.partial(
        pl.kernel,
        out_type=jax.ShapeDtypeStruct((T_STEPS * P_PAD, D_MODEL),
                                      jnp.float32),
        mesh=mesh,
        scratch_types=[
            [pltpu.VMEM((S_CHUNK,), jnp.int32)] * 2,
            [pltpu.VMEM((S_CHUNK,), jnp.int32)] * 2,
            [pltpu.VMEM((S_CHUNK, D_MODEL), jnp.float32)] * 2,
            [pltpu.SemaphoreType.DMA] * 2,
            [pltpu.SemaphoreType.DMA] * 2,
            [pltpu.SemaphoreType.DMA] * 2,
        ],
    )
    def sc_dispatch(x_hbm, sidx0_hbm, sidx1_hbm, out_hbm, i0_v, i1_v,
                    rows_v, lsem, s0sem, s1sem):
        wid = lax.axis_index("s") * 2 + lax.axis_index("c")
        base = wid * S_PER_W
        n_chunks = S_PER_W // S_CHUNK
        ld = [None, None]
        sc0 = [None, None]
        sc1 = [None, None]
        for c in range(n_chunks):
            b = c % 2
            o = base + c * S_CHUNK
            if sc0[b] is not None:
                sc0[b].wait()
                sc1[b].wait()
            pltpu.sync_copy(sidx0_hbm.at[pl.ds(o, S_CHUNK)], i0_v[b])
            pltpu.sync_copy(sidx1_hbm.at[pl.ds(o, S_CHUNK)], i1_v[b])
            ld[b] = pltpu.async_copy(
                x_hbm.at[pl.ds(o, S_CHUNK)], rows_v[b], lsem[b])
            if c > 0:
                p = 1 - b
                ld[p].wait()
                sc0[p] = pltpu.async_copy(
                    rows_v[p], out_hbm.at[i0_v[p]], s0sem[p])
                sc1[p] = pltpu.async_copy(
                    rows_v[p], out_hbm.at[i1_v[p]], s1sem[p])
        bl = (n_chunks - 1) % 2
        ld[bl].wait()
        pltpu.async_copy(rows_v[bl], out_hbm.at[i0_v[bl]], s0sem[bl]).wait()
        pltpu.async_copy(rows_v[bl], out_hbm.at[i1_v[bl]], s1sem[bl]).wait()
        if sc0[1 - bl] is not None:
            sc0[1 - bl].wait()
            sc1[1 - bl].wait()

    @functools.partial(
        pl.kernel,
        out_type=jax.ShapeDtypeStruct((2 * C_ROWS, D_MODEL), jnp.float32),
        mesh=mesh,
        scratch_types=[
            [pltpu.VMEM((2 * C_CHUNK,), jnp.int32)] * 2,
            [pltpu.VMEM((2 * C_CHUNK, D_MODEL), jnp.float32)] * 2,
            [pltpu.SemaphoreType.DMA] * 2,
            [pltpu.SemaphoreType.DMA] * 2,
        ],
    )
    def sc_combine(y_hbm, cidx_hbm, out_hbm, idx_v, rows_v, gsem, wsem):
        wid = lax.axis_index("s") * 2 + lax.axis_index("c")
        chunk = 2 * C_CHUNK
        base = wid * 2 * C_PER_W
        n_chunks = C_PER_W // C_CHUNK
        gd = [None, None]
        wd = [None, None]
        for c in range(n_chunks):
            b = c % 2
            o = base + c * chunk
            if wd[b] is not None:
                wd[b].wait()
            pltpu.sync_copy(cidx_hbm.at[pl.ds(o, chunk)], idx_v[b])
            gd[b] = pltpu.async_copy(y_hbm.at[idx_v[b]], rows_v[b], gsem[b])
            if c > 0:
                gd[1 - b].wait()
                wd[1 - b] = pltpu.async_copy(
                    rows_v[1 - b],
                    out_hbm.at[pl.ds(base + (c - 1) * chunk, chunk)],
                    wsem[1 - b])
        bl = (n_chunks - 1) % 2
        gd[bl].wait()
        pltpu.sync_copy(
            rows_v[bl],
            out_hbm.at[pl.ds(base + (n_chunks - 1) * chunk, chunk)])
        if wd[1 - bl] is not None:
            wd[1 - bl].wait()

    return sc_dispatch, sc_combine


def _add_body(ya_ref, yb_ref, w_ref, out_ref):
    w0 = w_ref[:, 0][:, None]
    w1 = w_ref[:, 1][:, None]
    out_ref[...] = ya_ref[0] * w0 + yb_ref[0] * w1


def _ffn_body(meta_ref, xg_ref, wup_ref, wdn_ref, y_ref, s_scratch):
    j = pl.program_id(0)
    ln = meta_ref[1, j]

    @pl.when(ln > 0)
    def _():
        xf = xg_ref[...].reshape(T_STEPS * PB, D_MODEL)
        h = lax.dot_general(
            xf, wup_ref[0], (((1,), (1,)), ((), ())),
            preferred_element_type=jnp.float32)

        v = jnp.zeros((PB, EXPERT_FF), jnp.float32)
        for t in range(T_STEPS):
            v = BETA * v + h[t * PB:(t + 1) * PB, :]
            s = (v >= V_TH).astype(jnp.float32)
            v = v - s * V_TH
            s_scratch[t * PB:(t + 1) * PB, :] = s

        o = lax.dot_general(
            s_scratch[...], wdn_ref[0], (((1,), (1,)), ((), ())),
            preferred_element_type=jnp.float32)

        v2 = jnp.zeros((PB, D_MODEL), jnp.float32)
        for t in range(T_STEPS):
            v2 = BETA * v2 + o[t * PB:(t + 1) * PB, :]
            s2 = (v2 >= V_TH).astype(jnp.float32)
            v2 = v2 - s2 * V_TH
            y_ref[t] = s2


def _routing_metadata(expert_indices):
    e_flat = expert_indices.reshape(-1).astype(jnp.int32)
    oh = (e_flat[:, None] ==
          jnp.arange(N_EXPERTS, dtype=jnp.int32)[None, :]).astype(jnp.int32)
    csum = jnp.cumsum(oh, axis=0)
    counts = csum[-1]
    local_rank = jnp.sum(oh * (csum - oh), axis=1)

    blocks_per_e = (counts + PB - 1) // PB
    blk_start = jnp.concatenate(
        [jnp.zeros((1,), jnp.int32),
         jnp.cumsum(blocks_per_e).astype(jnp.int32)])
    blk_start_p = jnp.sum(oh * blk_start[None, :N_EXPERTS], axis=1)
    slotp = blk_start_p * PB + local_rank

    n_blocks = blk_start[N_EXPERTS]
    j = jnp.arange(NBMAX, dtype=jnp.int32)
    j_eff = jnp.minimum(j, n_blocks - 1)
    blk_e = jnp.clip(
        jnp.searchsorted(blk_start, j_eff, side='right').astype(jnp.int32)
        - 1, 0, N_EXPERTS - 1)
    blk_len = jnp.clip(counts[blk_e] - (j_eff - blk_start[blk_e]) * PB,
                       0, PB)
    blk_len = jnp.where(j < n_blocks, blk_len, 0)
    meta = jnp.stack([blk_e, blk_len])

    tt = jnp.arange(T_STEPS, dtype=jnp.int32)[:, None] * P_PAD
    sidx0 = (tt + slotp[0::TOP_K][None, :]).reshape(-1)
    sidx1 = (tt + slotp[1::TOP_K][None, :]).reshape(-1)
    cidx = jnp.concatenate([sidx0, sidx1])
    return sidx0, sidx1, cidx, meta


def kernel(x, expert_indices, expert_weights, W_up, W_down):
    T, N, D = x.shape
    sidx0, sidx1, cidx, meta = _routing_metadata(expert_indices)

    sc_dispatch, sc_combine = _sc_kernels()
    xg = sc_dispatch(x.reshape(T * N, D), sidx0, sidx1)
    xg = xg.reshape(T_STEPS, P_PAD, D)

    grid_spec = pltpu.PrefetchScalarGridSpec(
        num_scalar_prefetch=1,
        grid=(NBMAX,),
        in_specs=[
            pl.BlockSpec((T_STEPS, PB, D_MODEL), lambda j, m: (0, j, 0)),
            pl.BlockSpec((1, EXPERT_FF, D_MODEL), lambda j, m: (m[0, j], 0, 0)),
            pl.BlockSpec((1, D_MODEL, EXPERT_FF), lambda j, m: (m[0, j], 0, 0)),
        ],
        out_specs=pl.BlockSpec((T_STEPS, PB, D_MODEL), lambda j, m: (0, j, 0)),
        scratch_shapes=[pltpu.VMEM((T_STEPS * PB, EXPERT_FF), jnp.float32)],
    )
    y = pl.pallas_call(
        _ffn_body,
        grid_spec=grid_spec,
        out_shape=jax.ShapeDtypeStruct((T_STEPS, P_PAD, D_MODEL), jnp.float32),
    )(meta, xg, W_up, W_down)

    y2 = sc_combine(y.reshape(T_STEPS * P_PAD, D), cidx)
    y2 = y2.reshape(2, C_ROWS, D)
    w_c = jnp.broadcast_to(expert_weights[None], (T_STEPS, N, TOP_K))
    w_c = w_c.reshape(C_ROWS, TOP_K)

    add_rows = 512
    out = pl.pallas_call(
        _add_body,
        grid=(C_ROWS // add_rows,),
        in_specs=[
            pl.BlockSpec((1, add_rows, D_MODEL), lambda i: (0, i, 0)),
            pl.BlockSpec((1, add_rows, D_MODEL), lambda i: (1, i, 0)),
            pl.BlockSpec((add_rows, TOP_K), lambda i: (i, 0)),
        ],
        out_specs=pl.BlockSpec((add_rows, D_MODEL), lambda i: (i, 0)),
        out_shape=jax.ShapeDtypeStruct((C_ROWS, D_MODEL), jnp.float32),
    )(y2, y2, w_c)
    return out.reshape(T, N, D)


# speedup vs baseline: 2.9595x; 1.1208x over previous
import functools

import jax
import jax.numpy as jnp
from jax import lax
from jax.experimental import pallas as pl
from jax.experimental.pallas import tpu as pltpu
from jax.experimental.pallas import tpu_sc as plsc

N_EXPERTS = 8
D_MODEL = 1024
EXPERT_FF = 512
TOP_K = 2
T_STEPS = 4
N_TOK = 2048
BETA = 0.9
V_TH = 1.0

N_PAIRS = N_TOK * TOP_K
PB = 256
NBMAX = N_PAIRS // PB + N_EXPERTS - 1
P_PAD = NBMAX * PB

NW = 32
S_ROWS = T_STEPS * N_TOK
S_PER_W = S_ROWS // NW
S_CHUNK = 32
C_ROWS = T_STEPS * N_TOK
C_PER_W = C_ROWS // NW
C_CHUNK = 16


@functools.lru_cache(maxsize=None)
def _sc_kernels():
    mesh = plsc.VectorSubcoreMesh(
        core_axis_name="c", subcore_axis_name="s", num_cores=2,
        num_subcores=16)

    @functools.partial(
        pl.kernel,
        out_type=jax.ShapeDtypeStruct((T_STEPS * P_PAD, D_MODEL),
                                      jnp.float32),
        mesh=mesh,
        scratch_types=[
            [pltpu.VMEM((S_CHUNK,), jnp.int32)] * 2,
            [pltpu.VMEM((S_CHUNK,), jnp.int32)] * 2,
            [pltpu.VMEM((S_CHUNK, D_MODEL), jnp.float32)] * 2,
            [pltpu.SemaphoreType.DMA] * 2,
            [pltpu.SemaphoreType.DMA] * 2,
            [pltpu.SemaphoreType.DMA] * 2,
        ],
    )
    def sc_dispatch(x_hbm, sidx0_hbm, sidx1_hbm, out_hbm, i0_v, i1_v,
                    rows_v, lsem, s0sem, s1sem):
        wid = lax.axis_index("s") * 2 + lax.axis_index("c")
        base = wid * S_PER_W
        n_chunks = S_PER_W // S_CHUNK
        ld = [None, None]
        sc0 = [None, None]
        sc1 = [None, None]
        for c in range(n_chunks):
            b = c % 2
            o = base + c * S_CHUNK
            if sc0[b] is not None:
                sc0[b].wait()
                sc1[b].wait()
            pltpu.sync_copy(sidx0_hbm.at[pl.ds(o, S_CHUNK)], i0_v[b])
            pltpu.sync_copy(sidx1_hbm.at[pl.ds(o, S_CHUNK)], i1_v[b])
            ld[b] = pltpu.async_copy(
                x_hbm.at[pl.ds(o, S_CHUNK)], rows_v[b], lsem[b])
            if c > 0:
                p = 1 - b
                ld[p].wait()
                sc0[p] = pltpu.async_copy(
                    rows_v[p], out_hbm.at[i0_v[p]], s0sem[p])
                sc1[p] = pltpu.async_copy(
                    rows_v[p], out_hbm.at[i1_v[p]], s1sem[p])
        bl = (n_chunks - 1) % 2
        ld[bl].wait()
        pltpu.async_copy(rows_v[bl], out_hbm.at[i0_v[bl]], s0sem[bl]).wait()
        pltpu.async_copy(rows_v[bl], out_hbm.at[i1_v[bl]], s1sem[bl]).wait()
        if sc0[1 - bl] is not None:
            sc0[1 - bl].wait()
            sc1[1 - bl].wait()

    @functools.partial(
        pl.kernel,
        out_type=jax.ShapeDtypeStruct((2 * C_ROWS, D_MODEL), jnp.float32),
        mesh=mesh,
        scratch_types=[
            [pltpu.VMEM((2 * C_CHUNK,), jnp.int32)] * 2,
            [pltpu.VMEM((2 * C_CHUNK, D_MODEL), jnp.float32)] * 2,
            [pltpu.SemaphoreType.DMA] * 2,
            [pltpu.SemaphoreType.DMA] * 2,
        ],
    )
    def sc_combine(y_hbm, cidx_hbm, out_hbm, idx_v, rows_v, gsem, wsem):
        wid = lax.axis_index("s") * 2 + lax.axis_index("c")
        chunk = 2 * C_CHUNK
        base = wid * 2 * C_PER_W
        n_chunks = C_PER_W // C_CHUNK
        gd = [None, None]
        wd = [None, None]
        for c in range(n_chunks):
            b = c % 2
            o = base + c * chunk
            if wd[b] is not None:
                wd[b].wait()
            pltpu.sync_copy(cidx_hbm.at[pl.ds(o, chunk)], idx_v[b])
            gd[b] = pltpu.async_copy(y_hbm.at[idx_v[b]], rows_v[b], gsem[b])
            if c > 0:
                gd[1 - b].wait()
                wd[1 - b] = pltpu.async_copy(
                    rows_v[1 - b],
                    out_hbm.at[pl.ds(base + (c - 1) * chunk, chunk)],
                    wsem[1 - b])
        bl = (n_chunks - 1) % 2
        gd[bl].wait()
        pltpu.sync_copy(
            rows_v[bl],
            out_hbm.at[pl.ds(base + (n_chunks - 1) * chunk, chunk)])
        if wd[1 - bl] is not None:
            wd[1 - bl].wait()

    return sc_dispatch, sc_combine


def _add_body(ya_ref, yb_ref, w_ref, out_ref):
    w0 = w_ref[:, 0][:, None]
    w1 = w_ref[:, 1][:, None]
    out_ref[...] = ya_ref[0] * w0 + yb_ref[0] * w1


def _ffn_body(meta_ref, xg_ref, wup_ref, wdn_ref, y_ref, s_scratch):
    j = pl.program_id(0)
    ln = meta_ref[1, j]

    @pl.when(ln > 0)
    def _():
        xf = xg_ref[...].reshape(T_STEPS * PB, D_MODEL)
        h = lax.dot_general(
            xf, wup_ref[0], (((1,), (1,)), ((), ())),
            preferred_element_type=jnp.float32)

        v = jnp.zeros((PB, EXPERT_FF), jnp.float32)
        for t in range(T_STEPS):
            v = BETA * v + h[t * PB:(t + 1) * PB, :]
            s = (v >= V_TH).astype(jnp.float32)
            v = v - s * V_TH
            s_scratch[t * PB:(t + 1) * PB, :] = s

        o = lax.dot_general(
            s_scratch[...], wdn_ref[0], (((1,), (1,)), ((), ())),
            preferred_element_type=jnp.float32)

        v2 = jnp.zeros((PB, D_MODEL), jnp.float32)
        for t in range(T_STEPS):
            v2 = BETA * v2 + o[t * PB:(t + 1) * PB, :]
            s2 = (v2 >= V_TH).astype(jnp.float32)
            v2 = v2 - s2 * V_TH
            y_ref[t] = s2


def _routing_metadata(expert_indices):
    e_flat = expert_indices.reshape(-1).astype(jnp.int32)
    oh = (e_flat[:, None] ==
          jnp.arange(N_EXPERTS, dtype=jnp.int32)[None, :]).astype(jnp.int32)
    csum = jnp.cumsum(oh, axis=0)
    counts = csum[-1]
    local_rank = jnp.sum(oh * (csum - oh), axis=1)

    blocks_per_e = (counts + PB - 1) // PB
    blk_start = jnp.concatenate(
        [jnp.zeros((1,), jnp.int32),
         jnp.cumsum(blocks_per_e).astype(jnp.int32)])
    blk_start_p = jnp.sum(oh * blk_start[None, :N_EXPERTS], axis=1)
    slotp = blk_start_p * PB + local_rank

    n_blocks = blk_start[N_EXPERTS]
    j = jnp.arange(NBMAX, dtype=jnp.int32)
    j_eff = jnp.minimum(j, n_blocks - 1)
    blk_e = jnp.clip(
        jnp.searchsorted(blk_start, j_eff, side='right').astype(jnp.int32)
        - 1, 0, N_EXPERTS - 1)
    blk_len = jnp.clip(counts[blk_e] - (j_eff - blk_start[blk_e]) * PB,
                       0, PB)
    blk_len = jnp.where(j < n_blocks, blk_len, 0)
    meta = jnp.stack([blk_e, blk_len])

    tt = jnp.arange(T_STEPS, dtype=jnp.int32)[:, None] * P_PAD
    sidx0 = (tt + slotp[0::TOP_K][None, :]).reshape(-1)
    sidx1 = (tt + slotp[1::TOP_K][None, :]).reshape(-1)
    cidx = jnp.concatenate([sidx0, sidx1])
    return sidx0, sidx1, cidx, meta


def kernel(x, expert_indices, expert_weights, W_up, W_down):
    T, N, D = x.shape
    sidx0, sidx1, cidx, meta = _routing_metadata(expert_indices)

    sc_dispatch, sc_combine = _sc_kernels()
    xg = sc_dispatch(x.reshape(T * N, D), sidx0, sidx1)
    xg = xg.reshape(T_STEPS, P_PAD, D)

    grid_spec = pltpu.PrefetchScalarGridSpec(
        num_scalar_prefetch=1,
        grid=(NBMAX,),
        in_specs=[
            pl.BlockSpec((T_STEPS, PB, D_MODEL), lambda j, m: (0, j, 0)),
            pl.BlockSpec((1, EXPERT_FF, D_MODEL), lambda j, m: (m[0, j], 0, 0)),
            pl.BlockSpec((1, D_MODEL, EXPERT_FF), lambda j, m: (m[0, j], 0, 0)),
        ],
        out_specs=pl.BlockSpec((T_STEPS, PB, D_MODEL), lambda j, m: (0, j, 0)),
        scratch_shapes=[pltpu.VMEM((T_STEPS * PB, EXPERT_FF), jnp.float32)],
    )
    y = pl.pallas_call(
        _ffn_body,
        grid_spec=grid_spec,
        out_shape=jax.ShapeDtypeStruct((T_STEPS, P_PAD, D_MODEL), jnp.float32),
    )(meta, xg, W_up, W_down)

    y2 = sc_combine(y.reshape(T_STEPS * P_PAD, D), cidx)
    y2 = y2.reshape(2, C_ROWS, D)
    w_c = jnp.broadcast_to(expert_weights[None], (T_STEPS, N, TOP_K))
    w_c = w_c.reshape(C_ROWS, TOP_K)

    add_rows = 512
    out = pl.pallas_call(
        _add_body,
        grid=(C_ROWS // add_rows,),
        in_specs=[
            pl.BlockSpec((1, add_rows, D_MODEL), lambda i: (0, i, 0)),
            pl.BlockSpec((1, add_rows, D_MODEL), lambda i: (1, i, 0)),
            pl.BlockSpec((add_rows, TOP_K), lambda i: (i, 0)),
        ],
        out_specs=pl.BlockSpec((add_rows, D_MODEL), lambda i: (i, 0)),
        out_shape=jax.ShapeDtypeStruct((C_ROWS, D_MODEL), jnp.float32),
    )(y2, y2, w_c)
    return out.reshape(T, N, D)
